# baseline (speedup 1.0000x reference)
import jax
import jax.numpy as jnp
from jax import lax
from jax.experimental import pallas as pl
from jax.experimental.pallas import tpu as pltpu
from jax.experimental.pallas import tpu_sc as plsc

N = 10000
E = 320000
R = 8
G = 32
D = 128
OUT = 8
RW = R + 1

NC = 2
NS = 16
NW = NC * NS
L = 16

EB = 128
NBAT = E // EB
NTB_W = 81
TBAT = NW * NTB_W
EP = TBAT * EB
M0 = 42
M1 = 120
CPT = 81920

EPT = E // NW
CH = 2000

BN = 1000
NBLK = N // BN

_mesh = plsc.VectorSubcoreMesh(core_axis_name="c", subcore_axis_name="s")
_sc_params = pltpu.CompilerParams(needs_layout_passes=False,
                                 use_tc_tiling_on_sc=False)


def _worker_ids():
    c = lax.axis_index("c")
    s = lax.axis_index("s")
    return c, s, c * NS + s


def _count_body(dst_ref, et_ref, out_ref, cnt_v, dbuf, ebuf):
    _, _, w = _worker_ids()

    def zero(i, carry):
        cnt_v[pl.ds(i * L, L)] = jnp.zeros((L,), jnp.float32)
        return carry

    lax.fori_loop(0, CPT // L, zero, 0)

    ones = jnp.ones((L,), jnp.float32)

    def chunk(ci, carry):
        base = w * EPT + ci * CH
        pltpu.sync_copy(dst_ref.at[pl.ds(base, CH)], dbuf)
        pltpu.sync_copy(et_ref.at[pl.ds(base, CH)], ebuf)

        def vec(j, c2):
            d16 = dbuf[pl.ds(j * L, L)]
            e16 = ebuf[pl.ds(j * L, L)]
            plsc.addupdate_scatter(cnt_v, [d16 * R + e16], ones)
            return c2

        lax.fori_loop(0, CH // L, vec, 0)
        return carry

    lax.fori_loop(0, EPT // CH, chunk, 0)
    pltpu.sync_copy(cnt_v, out_ref.at[w])


def _k1a(dst, edge_type):
    f = pl.kernel(
        _count_body,
        out_type=jax.ShapeDtypeStruct((NW, CPT), jnp.float32),
        mesh=_mesh,
        compiler_params=_sc_params,
        scratch_types=[
            pltpu.VMEM((CPT,), jnp.float32),
            pltpu.VMEM((CH,), jnp.int32),
            pltpu.VMEM((CH,), jnp.int32),
        ],
    )
    return f(dst, edge_type)


def _inv_body(cnt_ref, out_ref):
    s = jnp.sum(cnt_ref[...], axis=0)
    out_ref[...] = 1.0 / jnp.maximum(s, 1.0)


def _k1c(cnt_parts):
    cb = CPT // 128 // 10
    return pl.pallas_call(
        _inv_body,
        grid=(10,),
        in_specs=[pl.BlockSpec((NW, cb, 128), lambda i: (0, i, 0))],
        out_specs=pl.BlockSpec((cb, 128), lambda i: (i, 0)),
        out_shape=jax.ShapeDtypeStruct((CPT // 128, 128), jnp.float32),
    )(cnt_parts)


def _meta_body(src_ref, dst_ref, et_ref, inv_ref, gidx_ref, norm_ref,
               s2, d2, e2, n2, gsem):
    _, _, w = _worker_ids()
    b0 = w * NTB_W

    pltpu.sync_copy(src_ref.at[pl.ds(b0, NTB_W)], s2)
    pltpu.sync_copy(dst_ref.at[pl.ds(b0, NTB_W)], d2)
    pltpu.sync_copy(et_ref.at[pl.ds(b0, NTB_W)], e2)

    def rowfn(j, carry):
        for k in range(EB // L):
            sl = pl.ds(k * L, L)
            sv = s2[j, sl]
            dv = d2[j, sl]
            ev = e2[j, sl]
            s2[j, sl] = ev * N + sv
            d2[j, sl] = dv * R + ev
        return carry

    lax.fori_loop(0, NTB_W, rowfn, 0)
    pltpu.sync_copy(s2, gidx_ref.at[pl.ds(b0, NTB_W)])

    def fire(i, carry):
        pltpu.async_copy(inv_ref.at[d2.at[i]], n2.at[i], gsem)

        @pl.when(i >= 8)
        def _():
            pltpu.make_async_copy(inv_ref.at[d2.at[i - 8]], n2.at[i - 8],
                                  gsem).wait()

        return carry

    lax.fori_loop(0, NTB_W, fire, 0)
    for j in range(8):
        pltpu.make_async_copy(inv_ref.at[d2.at[NTB_W - 8 + j]],
                              n2.at[NTB_W - 8 + j], gsem).wait()

    jstart = jnp.clip(NBAT - b0, 0, NTB_W)

    def zrow(j, carry):
        for k in range(EB // L):
            n2[j, pl.ds(k * L, L)] = jnp.zeros((L,), jnp.float32)
        return carry

    lax.fori_loop(jstart, NTB_W, zrow, 0)
    pltpu.sync_copy(n2, norm_ref.at[pl.ds(b0, NTB_W)])


def _k1b(src2, dst2, et2, inv):
    f = pl.kernel(
        _meta_body,
        out_type=(
            jax.ShapeDtypeStruct((TBAT, EB), jnp.int32),
            jax.ShapeDtypeStruct((TBAT, EB), jnp.float32),
        ),
        mesh=_mesh,
        compiler_params=_sc_params,
        scratch_types=[
            pltpu.VMEM((NTB_W, EB), jnp.int32),
            pltpu.VMEM((NTB_W, EB), jnp.int32),
            pltpu.VMEM((NTB_W, EB), jnp.int32),
            pltpu.VMEM((NTB_W, EB), jnp.float32),
            pltpu.SemaphoreType.DMA,
        ],
    )
    return f(src2, dst2, et2, inv)


def _xmm_body(x_ref, w_ref, out_ref):
    out_ref[...] = jnp.dot(x_ref[...], w_ref[0],
                           preferred_element_type=jnp.float32)


def _k2(x, Ws):
    return pl.pallas_call(
        _xmm_body,
        grid=(NBLK, RW),
        in_specs=[
            pl.BlockSpec((BN, D), lambda nb, r: (nb, 0)),
            pl.BlockSpec((1, D, D), lambda nb, r: (r, 0, 0)),
        ],
        out_specs=pl.BlockSpec((BN, D), lambda nb, r: (r * NBLK + nb, 0)),
        out_shape=jax.ShapeDtypeStruct((RW * N, D), jnp.float32),
        compiler_params=pltpu.CompilerParams(
            dimension_semantics=("arbitrary", "arbitrary")),
    )(x, Ws)


def _agg_body(xall_ref, gidx_ref, norm_ref, dst_ref, zeros_ref, out_ref,
              acc, gi0, gi1, gi2b, nb0, nb1, nb2b, db0, db1, db2b,
              ds0, ds1, ds2, rows0, rows1, rows2,
              msem, gsem0, gsem1, gsem2, ssem):
    c, s, _ = _worker_ids()
    M = lax.select(c == 0, M0, M1)
    b0 = lax.select(c == 0, s * M0, NS * M0 + s * M1)
    gis = (gi0, gi1, gi2b)
    nbs = (nb0, nb1, nb2b)
    dbs = (db0, db1, db2b)
    dsc = (ds0, ds1, ds2)
    rws = (rows0, rows1, rows2)
    gsems = (gsem0, gsem1, gsem2)

    @pl.when(s == 0)
    def _():
        pltpu.sync_copy(zeros_ref, acc)

    def fire_meta(i, q):
        @pl.when(i < M)
        def _():
            pltpu.async_copy(gidx_ref.at[b0 + i], gis[q], msem)
            pltpu.async_copy(norm_ref.at[b0 + i], nbs[q], msem)
            pltpu.async_copy(dst_ref.at[b0 + i], dbs[q], msem)

    for q in range(3):
        fire_meta(q, q)
    plsc.subcore_barrier()

    def super_it(i0, carry):
        for p in range(3):
            i = i0 + p

            @pl.when(i < M)
            def _():
                pltpu.make_async_copy(gidx_ref.at[b0 + i], gis[p],
                                      msem).wait()
                pltpu.make_async_copy(norm_ref.at[b0 + i], nbs[p],
                                      msem).wait()
                pltpu.make_async_copy(dst_ref.at[b0 + i], dbs[p],
                                      msem).wait()

                @pl.when(i >= 3)
                def _():
                    pltpu.make_async_copy(rws[p], acc.at[dsc[p]],
                                          ssem).wait()

                pltpu.async_copy(xall_ref.at[gis[p]], rws[p], gsems[p])

            im1 = i - 1
            q = (p + 2) % 3

            @pl.when((im1 >= 0) & (im1 < M))
            def _():
                pltpu.make_async_copy(xall_ref.at[gis[q]], rws[q],
                                      gsems[q]).wait()
                for k in range(EB // L):
                    dsc[q][pl.ds(k * L, L)] = dbs[q][pl.ds(k * L, L)]

                def scale(j, c2):
                    nv = nbs[q][pl.ds(j * L, L)]
                    for k in range(L):
                        e = j * L + k
                        sc = nv[k]
                        for f in range(D // L):
                            rws[q][e, pl.ds(f * L, L)] = (
                                rws[q][e, pl.ds(f * L, L)] * sc)
                    return c2

                lax.fori_loop(0, EB // L, scale, 0)
                pltpu.async_copy(rws[q], acc.at[dsc[q]], ssem, add=True)
                fire_meta(im1 + 3, q)

        return carry

    lax.fori_loop(0, M // 3 + 1, lambda ii, cr: super_it(ii * 3, cr), 0)

    pltpu.make_async_copy(rws[0], acc.at[dsc[0]], ssem).wait()
    pltpu.make_async_copy(rws[1], acc.at[dsc[1]], ssem).wait()
    pltpu.make_async_copy(rws[2], acc.at[dsc[2]], ssem).wait()

    plsc.subcore_barrier()

    @pl.when(s < NBLK)
    def _():
        pltpu.sync_copy(acc.at[pl.ds(s * BN, BN)],
                        out_ref.at[c, pl.ds(s * BN, BN)])


def _agg(xall, gidx2, norm2, dst2, zeros_nd):
    f = pl.kernel(
        _agg_body,
        out_type=jax.ShapeDtypeStruct((NC, N, D), jnp.float32),
        mesh=_mesh,
        compiler_params=_sc_params,
        scratch_types=(
            [pltpu.VMEM_SHARED((N, D), jnp.float32)]
            + [pltpu.VMEM((EB,), jnp.int32) for _ in range(3)]
            + [pltpu.VMEM((EB,), jnp.float32) for _ in range(3)]
            + [pltpu.VMEM((EB,), jnp.int32) for _ in range(6)]
            + [pltpu.VMEM((EB, D), jnp.float32) for _ in range(3)]
            + [pltpu.SemaphoreType.DMA for _ in range(5)]
        ),
    )
    return f(xall, gidx2, norm2, dst2, zeros_nd)


def _l2_body(msg_ref, root_ref, b_ref, w_ref, out_ref, h_scr):
    r = pl.program_id(1)

    @pl.when(r == 0)
    def _():
        h_scr[...] = jnp.maximum(
            msg_ref[0] + msg_ref[1] + root_ref[...] + b_ref[...], 0.0)

    out_ref[...] = jnp.dot(h_scr[...], w_ref[0],
                           preferred_element_type=jnp.float32)


def _k4(msg1, xall1, b1r, W2s):
    return pl.pallas_call(
        _l2_body,
        grid=(NBLK, RW),
        in_specs=[
            pl.BlockSpec((NC, BN, D), lambda nb, r: (0, nb, 0)),
            pl.BlockSpec((BN, D), lambda nb, r: (R * NBLK + nb, 0)),
            pl.BlockSpec((1, D), lambda nb, r: (0, 0)),
            pl.BlockSpec((1, D, D), lambda nb, r: (r, 0, 0)),
        ],
        out_specs=pl.BlockSpec((BN, D), lambda nb, r: (r * NBLK + nb, 0)),
        out_shape=jax.ShapeDtypeStruct((RW * N, D), jnp.float32),
        scratch_shapes=[pltpu.VMEM((BN, D), jnp.float32)],
        compiler_params=pltpu.CompilerParams(
            dimension_semantics=("arbitrary", "arbitrary")),
    )(msg1, xall1, b1r, W2s)


def _fin_body(msg_ref, root_ref, b_ref, batch_ref, emb_ref, sums_ref,
              cnts_ref):
    nb = pl.program_id(0)
    emb = jnp.maximum(
        msg_ref[0] + msg_ref[1] + root_ref[...] + b_ref[...], 0.0)
    emb_ref[...] = emb
    bt = batch_ref[0, 0, :]
    onehot = (bt[None, :] ==
              lax.broadcasted_iota(jnp.int32, (G, BN), 0)).astype(jnp.float32)
    ps = jnp.dot(onehot, emb, preferred_element_type=jnp.float32)
    pc = jnp.sum(onehot, axis=1, keepdims=True) * jnp.ones((1, D), jnp.float32)

    @pl.when(nb == 0)
    def _():
        sums_ref[...] = jnp.zeros_like(sums_ref)
        cnts_ref[...] = jnp.zeros_like(cnts_ref)

    sums_ref[...] += ps
    cnts_ref[...] += pc


def _k6(msg2, xall2, b2r, batch3):
    return pl.pallas_call(
        _fin_body,
        grid=(NBLK,),
        in_specs=[
            pl.BlockSpec((NC, BN, D), lambda nb: (0, nb, 0)),
            pl.BlockSpec((BN, D), lambda nb: (R * NBLK + nb, 0)),
            pl.BlockSpec((1, D), lambda nb: (0, 0)),
            pl.BlockSpec((1, 1, BN), lambda nb: (nb, 0, 0)),
        ],
        out_specs=[
            pl.BlockSpec((BN, D), lambda nb: (nb, 0)),
            pl.BlockSpec((G, D), lambda nb: (0, 0)),
            pl.BlockSpec((G, D), lambda nb: (0, 0)),
        ],
        out_shape=[
            jax.ShapeDtypeStruct((N, D), jnp.float32),
            jax.ShapeDtypeStruct((G, D), jnp.float32),
            jax.ShapeDtypeStruct((G, D), jnp.float32),
        ],
        compiler_params=pltpu.CompilerParams(
            dimension_semantics=("arbitrary",)),
    )(msg2, xall2, b2r, batch3)


def _mlp_body(sums_ref, cnts_ref, wm1_ref, bm1_ref, wm2_ref, bm2_ref,
              logits_ref, probs_ref):
    pooled = sums_ref[...] / jnp.maximum(cnts_ref[...], 1.0)
    v = jnp.dot(pooled, wm1_ref[...],
                preferred_element_type=jnp.float32) + bm1_ref[...]
    hm = jnp.where(v > 0, v, jnp.exp(jnp.minimum(v, 0.0)) - 1.0)
    lg = jnp.dot(hm, wm2_ref[...],
                 preferred_element_type=jnp.float32) + bm2_ref[...]
    logits_ref[...] = lg
    colmask = lax.broadcasted_iota(jnp.int32, (G, D), 1) < OUT
    lgm = jnp.where(colmask, lg, -1e30)
    m = jnp.max(lgm, axis=1, keepdims=True)
    ex = jnp.where(colmask, jnp.exp(lgm - m), 0.0)
    probs_ref[...] = ex / jnp.sum(ex, axis=1, keepdims=True)


def _k7(sums, cnts, Wm1, bm1r, Wm2p, bm2p):
    return pl.pallas_call(
        _mlp_body,
        out_shape=[
            jax.ShapeDtypeStruct((G, D), jnp.float32),
            jax.ShapeDtypeStruct((G, D), jnp.float32),
        ],
    )(sums, cnts, Wm1, bm1r, Wm2p, bm2p)


def kernel(x, W1, root1, b1, W2, root2, b2, Wm1, bm1, Wm2, bm2,
           edge_index, edge_type, batch):
    W1s = jnp.concatenate([W1, root1[None]], axis=0)
    W2s = jnp.concatenate([W2, root2[None]], axis=0)
    zeros_nd = jnp.zeros((N, D), jnp.float32)
    b1r = b1.reshape(1, D)
    b2r = b2.reshape(1, D)
    batch3 = batch.reshape(NBLK, 1, BN)
    Wm2p = jnp.pad(Wm2, ((0, 0), (0, D - OUT)))
    bm2p = jnp.pad(bm2, (0, D - OUT)).reshape(1, D)

    src = edge_index[0]
    dst = edge_index[1]
    pad = EP - E
    src2 = jnp.pad(src, (0, pad)).reshape(TBAT, EB)
    dst2 = jnp.pad(dst, (0, pad)).reshape(TBAT, EB)
    et2 = jnp.pad(edge_type, (0, pad)).reshape(TBAT, EB)

    cnt_parts = _k1a(dst, edge_type)
    inv = _k1c(cnt_parts.reshape(NW, CPT // 128, 128)).reshape(CPT)
    gidx2, norm2 = _k1b(src2, dst2, et2, inv)

    xall1 = _k2(x, W1s)
    msg1 = _agg(xall1, gidx2, norm2, dst2, zeros_nd)
    xall2 = _k4(msg1, xall1, b1r, W2s)
    msg2 = _agg(xall2, gidx2, norm2, dst2, zeros_nd)
    emb, sums, cnts = _k6(msg2, xall2, b2r, batch3)
    logits_p, probs_p = _k7(sums, cnts, Wm1, bm1.reshape(1, D), Wm2p, bm2p)
    return (logits_p[:, :OUT], probs_p[:, :OUT], emb)


# speedup vs baseline: 3.2536x; 3.2536x over previous
import jax
import jax.numpy as jnp
from jax import lax
from jax.experimental import pallas as pl
from jax.experimental.pallas import tpu as pltpu
from jax.experimental.pallas import tpu_sc as plsc

N = 10000
E = 320000
R = 8
G = 32
D = 128
OUT = 8
RW = R + 1

NC = 2
NS = 16
NW = NC * NS
L = 16

EB = 128
NBAT = E // EB
NTB_W = 81
TBAT = NW * NTB_W
EP = TBAT * EB
M0 = 81
M1 = 81
CPT = 81920

EPT = E // NW
CH = 2000

BN = 1000
NBLK = N // BN

_mesh = plsc.VectorSubcoreMesh(core_axis_name="c", subcore_axis_name="s")
_sc_params = pltpu.CompilerParams(needs_layout_passes=False,
                                 use_tc_tiling_on_sc=False)


def _worker_ids():
    c = lax.axis_index("c")
    s = lax.axis_index("s")
    return c, s, c * NS + s


def _count_body(dst_ref, et_ref, out_ref, cnt_v, dbuf, ebuf):
    _, _, w = _worker_ids()

    def zero(i, carry):
        cnt_v[pl.ds(i * L, L)] = jnp.zeros((L,), jnp.float32)
        return carry

    lax.fori_loop(0, CPT // L, zero, 0)

    ones = jnp.ones((L,), jnp.float32)

    def chunk(ci, carry):
        base = w * EPT + ci * CH
        pltpu.sync_copy(dst_ref.at[pl.ds(base, CH)], dbuf)
        pltpu.sync_copy(et_ref.at[pl.ds(base, CH)], ebuf)

        def vec(j, c2):
            d16 = dbuf[pl.ds(j * L, L)]
            e16 = ebuf[pl.ds(j * L, L)]
            plsc.addupdate_scatter(cnt_v, [d16 * R + e16], ones)
            return c2

        lax.fori_loop(0, CH // L, vec, 0)
        return carry

    lax.fori_loop(0, EPT // CH, chunk, 0)
    pltpu.sync_copy(cnt_v, out_ref.at[w])


def _k1a(dst, edge_type):
    f = pl.kernel(
        _count_body,
        out_type=jax.ShapeDtypeStruct((NW, CPT), jnp.float32),
        mesh=_mesh,
        compiler_params=_sc_params,
        scratch_types=[
            pltpu.VMEM((CPT,), jnp.float32),
            pltpu.VMEM((CH,), jnp.int32),
            pltpu.VMEM((CH,), jnp.int32),
        ],
    )
    return f(dst, edge_type)


def _inv_body(cnt_ref, out_ref):
    s = jnp.sum(cnt_ref[...], axis=0)
    out_ref[...] = 1.0 / jnp.maximum(s, 1.0)


def _k1c(cnt_parts):
    cb = CPT // 128 // 10
    return pl.pallas_call(
        _inv_body,
        grid=(10,),
        in_specs=[pl.BlockSpec((NW, cb, 128), lambda i: (0, i, 0))],
        out_specs=pl.BlockSpec((cb, 128), lambda i: (i, 0)),
        out_shape=jax.ShapeDtypeStruct((CPT // 128, 128), jnp.float32),
    )(cnt_parts)


def _meta_body(src_ref, dst_ref, et_ref, inv_ref, gidx_ref, norm_ref,
               s2, d2, e2, n2, gsem):
    _, _, w = _worker_ids()
    b0 = w * NTB_W

    pltpu.sync_copy(src_ref.at[pl.ds(b0, NTB_W)], s2)
    pltpu.sync_copy(dst_ref.at[pl.ds(b0, NTB_W)], d2)
    pltpu.sync_copy(et_ref.at[pl.ds(b0, NTB_W)], e2)

    def rowfn(j, carry):
        for k in range(EB // L):
            sl = pl.ds(k * L, L)
            sv = s2[j, sl]
            dv = d2[j, sl]
            ev = e2[j, sl]
            s2[j, sl] = ev * N + sv
            d2[j, sl] = dv * R + ev
        return carry

    lax.fori_loop(0, NTB_W, rowfn, 0)
    pltpu.sync_copy(s2, gidx_ref.at[pl.ds(b0, NTB_W)])

    def fire(i, carry):
        pltpu.async_copy(inv_ref.at[d2.at[i]], n2.at[i], gsem)

        @pl.when(i >= 8)
        def _():
            pltpu.make_async_copy(inv_ref.at[d2.at[i - 8]], n2.at[i - 8],
                                  gsem).wait()

        return carry

    lax.fori_loop(0, NTB_W, fire, 0)
    for j in range(8):
        pltpu.make_async_copy(inv_ref.at[d2.at[NTB_W - 8 + j]],
                              n2.at[NTB_W - 8 + j], gsem).wait()

    jstart = jnp.clip(NBAT - b0, 0, NTB_W)

    def zrow(j, carry):
        for k in range(EB // L):
            n2[j, pl.ds(k * L, L)] = jnp.zeros((L,), jnp.float32)
        return carry

    lax.fori_loop(jstart, NTB_W, zrow, 0)
    pltpu.sync_copy(n2, norm_ref.at[pl.ds(b0, NTB_W)])


def _k1b(src2, dst2, et2, inv):
    f = pl.kernel(
        _meta_body,
        out_type=(
            jax.ShapeDtypeStruct((TBAT, EB), jnp.int32),
            jax.ShapeDtypeStruct((TBAT, EB), jnp.float32),
        ),
        mesh=_mesh,
        compiler_params=_sc_params,
        scratch_types=[
            pltpu.VMEM((NTB_W, EB), jnp.int32),
            pltpu.VMEM((NTB_W, EB), jnp.int32),
            pltpu.VMEM((NTB_W, EB), jnp.int32),
            pltpu.VMEM((NTB_W, EB), jnp.float32),
            pltpu.SemaphoreType.DMA,
        ],
    )
    return f(src2, dst2, et2, inv)


def _xmm_body(x_ref, w_ref, out_ref):
    out_ref[...] = jnp.dot(x_ref[...], w_ref[0],
                           preferred_element_type=jnp.float32)


def _k2(x, Ws):
    return pl.pallas_call(
        _xmm_body,
        grid=(NBLK, RW),
        in_specs=[
            pl.BlockSpec((BN, D), lambda nb, r: (nb, 0)),
            pl.BlockSpec((1, D, D), lambda nb, r: (r, 0, 0)),
        ],
        out_specs=pl.BlockSpec((BN, D), lambda nb, r: (r * NBLK + nb, 0)),
        out_shape=jax.ShapeDtypeStruct((RW * N, D), jnp.float32),
        compiler_params=pltpu.CompilerParams(
            dimension_semantics=("arbitrary", "arbitrary")),
    )(x, Ws)


def _agg_body(xall_ref, gidx_ref, norm_ref, dst_ref, zeros_ref, out_ref,
              acc, gi0, gi1, gi2b, nb0, nb1, nb2b, db0, db1, db2b,
              ds0, ds1, ds2, rows0, rows1, rows2,
              msem, gsem0, gsem1, gsem2, ssem):
    c, s, _ = _worker_ids()
    M = lax.select(c == 0, M0, M1)
    b0 = lax.select(c == 0, s * M0, NS * M0 + s * M1)
    gis = (gi0, gi1, gi2b)
    nbs = (nb0, nb1, nb2b)
    dbs = (db0, db1, db2b)
    dsc = (ds0, ds1, ds2)
    rws = (rows0, rows1, rows2)
    gsems = (gsem0, gsem1, gsem2)

    @pl.when(s == 0)
    def _():
        pltpu.sync_copy(zeros_ref, acc)

    def fire_meta(i, q):
        @pl.when(i < M)
        def _():
            pltpu.async_copy(gidx_ref.at[b0 + i], gis[q], msem)
            pltpu.async_copy(norm_ref.at[b0 + i], nbs[q], msem)
            pltpu.async_copy(dst_ref.at[b0 + i], dbs[q], msem)

    for q in range(3):
        fire_meta(q, q)
    plsc.subcore_barrier()

    def super_it(i0, carry):
        for p in range(3):
            i = i0 + p

            @pl.when(i < M)
            def _():
                pltpu.make_async_copy(gidx_ref.at[b0 + i], gis[p],
                                      msem).wait()
                pltpu.make_async_copy(norm_ref.at[b0 + i], nbs[p],
                                      msem).wait()
                pltpu.make_async_copy(dst_ref.at[b0 + i], dbs[p],
                                      msem).wait()

                @pl.when(i >= 3)
                def _():
                    pltpu.make_async_copy(rws[p], acc.at[dsc[p]],
                                          ssem).wait()

                pltpu.async_copy(xall_ref.at[gis[p]], rws[p], gsems[p])

            im1 = i - 1
            q = (p + 2) % 3

            @pl.when((im1 >= 0) & (im1 < M))
            def _():
                pltpu.make_async_copy(xall_ref.at[gis[q]], rws[q],
                                      gsems[q]).wait()
                for k in range(EB // L):
                    dsc[q][pl.ds(k * L, L)] = dbs[q][pl.ds(k * L, L)]

                def scale(j, c2):
                    nv = nbs[q][pl.ds(j * L, L)]
                    for k in range(L):
                        e = j * L + k
                        sc = nv[k]
                        for f in range(D // L):
                            rws[q][e, pl.ds(f * L, L)] = (
                                rws[q][e, pl.ds(f * L, L)] * sc)
                    return c2

                lax.fori_loop(0, EB // L, scale, 0)
                pltpu.async_copy(rws[q], acc.at[dsc[q]], ssem, add=True)
                fire_meta(im1 + 3, q)

        return carry

    lax.fori_loop(0, M // 3 + 1, lambda ii, cr: super_it(ii * 3, cr), 0)

    pltpu.make_async_copy(rws[0], acc.at[dsc[0]], ssem).wait()
    pltpu.make_async_copy(rws[1], acc.at[dsc[1]], ssem).wait()
    pltpu.make_async_copy(rws[2], acc.at[dsc[2]], ssem).wait()

    plsc.subcore_barrier()

    @pl.when(s < NBLK)
    def _():
        pltpu.sync_copy(acc.at[pl.ds(s * BN, BN)],
                        out_ref.at[c, pl.ds(s * BN, BN)])


def _agg(xall, gidx2, norm2, dst2, zeros_nd):
    f = pl.kernel(
        _agg_body,
        out_type=jax.ShapeDtypeStruct((NC, N, D), jnp.float32),
        mesh=_mesh,
        compiler_params=_sc_params,
        scratch_types=(
            [pltpu.VMEM_SHARED((N, D), jnp.float32)]
            + [pltpu.VMEM((EB,), jnp.int32) for _ in range(3)]
            + [pltpu.VMEM((EB,), jnp.float32) for _ in range(3)]
            + [pltpu.VMEM((EB,), jnp.int32) for _ in range(6)]
            + [pltpu.VMEM((EB, D), jnp.float32) for _ in range(3)]
            + [pltpu.SemaphoreType.DMA for _ in range(5)]
        ),
    )
    return f(xall, gidx2, norm2, dst2, zeros_nd)


def _l2_body(msg_ref, root_ref, b_ref, w_ref, out_ref, h_scr):
    r = pl.program_id(1)

    @pl.when(r == 0)
    def _():
        h_scr[...] = jnp.maximum(
            msg_ref[0] + msg_ref[1] + root_ref[...] + b_ref[...], 0.0)

    out_ref[...] = jnp.dot(h_scr[...], w_ref[0],
                           preferred_element_type=jnp.float32)


def _k4(msg1, xall1, b1r, W2s):
    return pl.pallas_call(
        _l2_body,
        grid=(NBLK, RW),
        in_specs=[
            pl.BlockSpec((NC, BN, D), lambda nb, r: (0, nb, 0)),
            pl.BlockSpec((BN, D), lambda nb, r: (R * NBLK + nb, 0)),
            pl.BlockSpec((1, D), lambda nb, r: (0, 0)),
            pl.BlockSpec((1, D, D), lambda nb, r: (r, 0, 0)),
        ],
        out_specs=pl.BlockSpec((BN, D), lambda nb, r: (r * NBLK + nb, 0)),
        out_shape=jax.ShapeDtypeStruct((RW * N, D), jnp.float32),
        scratch_shapes=[pltpu.VMEM((BN, D), jnp.float32)],
        compiler_params=pltpu.CompilerParams(
            dimension_semantics=("arbitrary", "arbitrary")),
    )(msg1, xall1, b1r, W2s)


def _fin_body(msg_ref, root_ref, b_ref, batch_ref, emb_ref, sums_ref,
              cnts_ref):
    nb = pl.program_id(0)
    emb = jnp.maximum(
        msg_ref[0] + msg_ref[1] + root_ref[...] + b_ref[...], 0.0)
    emb_ref[...] = emb
    bt = batch_ref[0, 0, :]
    onehot = (bt[None, :] ==
              lax.broadcasted_iota(jnp.int32, (G, BN), 0)).astype(jnp.float32)
    ps = jnp.dot(onehot, emb, preferred_element_type=jnp.float32)
    pc = jnp.sum(onehot, axis=1, keepdims=True) * jnp.ones((1, D), jnp.float32)

    @pl.when(nb == 0)
    def _():
        sums_ref[...] = jnp.zeros_like(sums_ref)
        cnts_ref[...] = jnp.zeros_like(cnts_ref)

    sums_ref[...] += ps
    cnts_ref[...] += pc


def _k6(msg2, xall2, b2r, batch3):
    return pl.pallas_call(
        _fin_body,
        grid=(NBLK,),
        in_specs=[
            pl.BlockSpec((NC, BN, D), lambda nb: (0, nb, 0)),
            pl.BlockSpec((BN, D), lambda nb: (R * NBLK + nb, 0)),
            pl.BlockSpec((1, D), lambda nb: (0, 0)),
            pl.BlockSpec((1, 1, BN), lambda nb: (nb, 0, 0)),
        ],
        out_specs=[
            pl.BlockSpec((BN, D), lambda nb: (nb, 0)),
            pl.BlockSpec((G, D), lambda nb: (0, 0)),
            pl.BlockSpec((G, D), lambda nb: (0, 0)),
        ],
        out_shape=[
            jax.ShapeDtypeStruct((N, D), jnp.float32),
            jax.ShapeDtypeStruct((G, D), jnp.float32),
            jax.ShapeDtypeStruct((G, D), jnp.float32),
        ],
        compiler_params=pltpu.CompilerParams(
            dimension_semantics=("arbitrary",)),
    )(msg2, xall2, b2r, batch3)


def _mlp_body(sums_ref, cnts_ref, wm1_ref, bm1_ref, wm2_ref, bm2_ref,
              logits_ref, probs_ref):
    pooled = sums_ref[...] / jnp.maximum(cnts_ref[...], 1.0)
    v = jnp.dot(pooled, wm1_ref[...],
                preferred_element_type=jnp.float32) + bm1_ref[...]
    hm = jnp.where(v > 0, v, jnp.exp(jnp.minimum(v, 0.0)) - 1.0)
    lg = jnp.dot(hm, wm2_ref[...],
                 preferred_element_type=jnp.float32) + bm2_ref[...]
    logits_ref[...] = lg
    colmask = lax.broadcasted_iota(jnp.int32, (G, D), 1) < OUT
    lgm = jnp.where(colmask, lg, -1e30)
    m = jnp.max(lgm, axis=1, keepdims=True)
    ex = jnp.where(colmask, jnp.exp(lgm - m), 0.0)
    probs_ref[...] = ex / jnp.sum(ex, axis=1, keepdims=True)


def _k7(sums, cnts, Wm1, bm1r, Wm2p, bm2p):
    return pl.pallas_call(
        _mlp_body,
        out_shape=[
            jax.ShapeDtypeStruct((G, D), jnp.float32),
            jax.ShapeDtypeStruct((G, D), jnp.float32),
        ],
    )(sums, cnts, Wm1, bm1r, Wm2p, bm2p)


def kernel(x, W1, root1, b1, W2, root2, b2, Wm1, bm1, Wm2, bm2,
           edge_index, edge_type, batch):
    W1s = jnp.concatenate([W1, root1[None]], axis=0)
    W2s = jnp.concatenate([W2, root2[None]], axis=0)
    zeros_nd = jnp.zeros((N, D), jnp.float32)
    b1r = b1.reshape(1, D)
    b2r = b2.reshape(1, D)
    batch3 = batch.reshape(NBLK, 1, BN)
    Wm2p = jnp.pad(Wm2, ((0, 0), (0, D - OUT)))
    bm2p = jnp.pad(bm2, (0, D - OUT)).reshape(1, D)

    src = edge_index[0]
    dst = edge_index[1]
    pad = EP - E
    src2 = jnp.pad(src, (0, pad)).reshape(TBAT, EB)
    dst2 = jnp.pad(dst, (0, pad)).reshape(TBAT, EB)
    et2 = jnp.pad(edge_type, (0, pad)).reshape(TBAT, EB)

    cnt_parts = _k1a(dst, edge_type)
    inv = _k1c(cnt_parts.reshape(NW, CPT // 128, 128)).reshape(CPT)
    gidx2, norm2 = _k1b(src2, dst2, et2, inv)

    xall1 = _k2(x, W1s)
    msg1 = _agg(xall1, gidx2, norm2, dst2, zeros_nd)
    xall2 = _k4(msg1, xall1, b1r, W2s)
    msg2 = _agg(xall2, gidx2, norm2, dst2, zeros_nd)
    emb, sums, cnts = _k6(msg2, xall2, b2r, batch3)
    logits_p, probs_p = _k7(sums, cnts, Wm1, bm1.reshape(1, D), Wm2p, bm2p)
    return (logits_p[:, :OUT], probs_p[:, :OUT], emb)


# speedup vs baseline: 3.2568x; 1.0010x over previous
import jax
import jax.numpy as jnp
from jax import lax
from jax.experimental import pallas as pl
from jax.experimental.pallas import tpu as pltpu
from jax.experimental.pallas import tpu_sc as plsc

N = 10000
E = 320000
R = 8
G = 32
D = 128
OUT = 8
RW = R + 1

NC = 2
NS = 16
NW = NC * NS
L = 16

EB = 128
NBAT = E // EB
NTB_W = 81
TBAT = NW * NTB_W
EP = TBAT * EB
M0 = 81
M1 = 81
CPT = 81920

EPT = E // NW
CH = 2000

BN = 1000
NBLK = N // BN

_mesh = plsc.VectorSubcoreMesh(core_axis_name="c", subcore_axis_name="s")
_sc_params = pltpu.CompilerParams(needs_layout_passes=False,
                                 use_tc_tiling_on_sc=False)


def _worker_ids():
    c = lax.axis_index("c")
    s = lax.axis_index("s")
    return c, s, c * NS + s


def _count_body(dst_ref, et_ref, out_ref, cnt_v, dbuf, ebuf):
    _, _, w = _worker_ids()

    def zero(i, carry):
        cnt_v[pl.ds(i * L, L)] = jnp.zeros((L,), jnp.float32)
        return carry

    lax.fori_loop(0, CPT // L, zero, 0)

    ones = jnp.ones((L,), jnp.float32)

    def chunk(ci, carry):
        base = w * EPT + ci * CH
        pltpu.sync_copy(dst_ref.at[pl.ds(base, CH)], dbuf)
        pltpu.sync_copy(et_ref.at[pl.ds(base, CH)], ebuf)

        def vec(j, c2):
            d16 = dbuf[pl.ds(j * L, L)]
            e16 = ebuf[pl.ds(j * L, L)]
            plsc.addupdate_scatter(cnt_v, [d16 * R + e16], ones)
            return c2

        lax.fori_loop(0, CH // L, vec, 0)
        return carry

    lax.fori_loop(0, EPT // CH, chunk, 0)
    pltpu.sync_copy(cnt_v, out_ref.at[w])


def _k1a(dst, edge_type):
    f = pl.kernel(
        _count_body,
        out_type=jax.ShapeDtypeStruct((NW, CPT), jnp.float32),
        mesh=_mesh,
        compiler_params=_sc_params,
        scratch_types=[
            pltpu.VMEM((CPT,), jnp.float32),
            pltpu.VMEM((CH,), jnp.int32),
            pltpu.VMEM((CH,), jnp.int32),
        ],
    )
    return f(dst, edge_type)


def _inv_body(cnt_ref, out_ref):
    s = jnp.sum(cnt_ref[...], axis=0)
    out_ref[...] = 1.0 / jnp.maximum(s, 1.0)


def _k1c(cnt_parts):
    cb = CPT // 128 // 10
    return pl.pallas_call(
        _inv_body,
        grid=(10,),
        in_specs=[pl.BlockSpec((NW, cb, 128), lambda i: (0, i, 0))],
        out_specs=pl.BlockSpec((cb, 128), lambda i: (i, 0)),
        out_shape=jax.ShapeDtypeStruct((CPT // 128, 128), jnp.float32),
    )(cnt_parts)


def _meta_body(src_ref, dst_ref, et_ref, inv_ref, gidx_ref, norm_ref,
               s2, d2, e2, n2, gsem):
    _, _, w = _worker_ids()
    b0 = w * NTB_W

    pltpu.sync_copy(src_ref.at[pl.ds(b0, NTB_W)], s2)
    pltpu.sync_copy(dst_ref.at[pl.ds(b0, NTB_W)], d2)
    pltpu.sync_copy(et_ref.at[pl.ds(b0, NTB_W)], e2)

    def rowfn(j, carry):
        for k in range(EB // L):
            sl = pl.ds(k * L, L)
            sv = s2[j, sl]
            dv = d2[j, sl]
            ev = e2[j, sl]
            s2[j, sl] = ev * N + sv
            d2[j, sl] = dv * R + ev
        return carry

    lax.fori_loop(0, NTB_W, rowfn, 0)
    pltpu.sync_copy(s2, gidx_ref.at[pl.ds(b0, NTB_W)])

    def fire(i, carry):
        pltpu.async_copy(inv_ref.at[d2.at[i]], n2.at[i], gsem)

        @pl.when(i >= 8)
        def _():
            pltpu.make_async_copy(inv_ref.at[d2.at[i - 8]], n2.at[i - 8],
                                  gsem).wait()

        return carry

    lax.fori_loop(0, NTB_W, fire, 0)
    for j in range(8):
        pltpu.make_async_copy(inv_ref.at[d2.at[NTB_W - 8 + j]],
                              n2.at[NTB_W - 8 + j], gsem).wait()

    jstart = jnp.clip(NBAT - b0, 0, NTB_W)

    def zrow(j, carry):
        for k in range(EB // L):
            n2[j, pl.ds(k * L, L)] = jnp.zeros((L,), jnp.float32)
        return carry

    lax.fori_loop(jstart, NTB_W, zrow, 0)
    pltpu.sync_copy(n2, norm_ref.at[pl.ds(b0, NTB_W)])


def _k1b(src2, dst2, et2, inv):
    f = pl.kernel(
        _meta_body,
        out_type=(
            jax.ShapeDtypeStruct((TBAT, EB), jnp.int32),
            jax.ShapeDtypeStruct((TBAT, EB), jnp.float32),
        ),
        mesh=_mesh,
        compiler_params=_sc_params,
        scratch_types=[
            pltpu.VMEM((NTB_W, EB), jnp.int32),
            pltpu.VMEM((NTB_W, EB), jnp.int32),
            pltpu.VMEM((NTB_W, EB), jnp.int32),
            pltpu.VMEM((NTB_W, EB), jnp.float32),
            pltpu.SemaphoreType.DMA,
        ],
    )
    return f(src2, dst2, et2, inv)


def _xmm_body(x_ref, w_ref, out_ref):
    out_ref[...] = jnp.dot(x_ref[...], w_ref[0],
                           preferred_element_type=jnp.float32)


def _k2(x, Ws):
    return pl.pallas_call(
        _xmm_body,
        grid=(NBLK, RW),
        in_specs=[
            pl.BlockSpec((BN, D), lambda nb, r: (nb, 0)),
            pl.BlockSpec((1, D, D), lambda nb, r: (r, 0, 0)),
        ],
        out_specs=pl.BlockSpec((BN, D), lambda nb, r: (r * NBLK + nb, 0)),
        out_shape=jax.ShapeDtypeStruct((RW * N, D), jnp.float32),
        compiler_params=pltpu.CompilerParams(
            dimension_semantics=("arbitrary", "arbitrary")),
    )(x, Ws)


def _agg_body(xall_ref, gidx_ref, norm_ref, dst_ref, zeros_ref, out_ref,
              acc, gi0, gi1, gi2b, nb0, nb1, nb2b, db0, db1, db2b,
              ds0, ds1, ds2, rows0, rows1, rows2,
              msem, gsem0, gsem1, gsem2, ssem):
    c, s, _ = _worker_ids()
    M = lax.select(c == 0, M0, M1)
    b0 = lax.select(c == 0, s * M0, NS * M0 + s * M1)
    gis = (gi0, gi1, gi2b)
    nbs = (nb0, nb1, nb2b)
    dbs = (db0, db1, db2b)
    dsc = (ds0, ds1, ds2)
    rws = (rows0, rows1, rows2)
    gsems = (gsem0, gsem1, gsem2)

    @pl.when(s == 0)
    def _():
        pltpu.sync_copy(zeros_ref, acc)

    def fire_meta(i, q):
        @pl.when(i < M)
        def _():
            pltpu.async_copy(gidx_ref.at[b0 + i], gis[q], msem)
            pltpu.async_copy(norm_ref.at[b0 + i], nbs[q], msem)
            pltpu.async_copy(dst_ref.at[b0 + i], dbs[q], msem)

    for q in range(3):
        fire_meta(q, q)
    plsc.subcore_barrier()

    def super_it(i0, carry):
        for p in range(3):
            i = i0 + p

            @pl.when(i < M)
            def _():
                pltpu.make_async_copy(gidx_ref.at[b0 + i], gis[p],
                                      msem).wait()
                pltpu.make_async_copy(norm_ref.at[b0 + i], nbs[p],
                                      msem).wait()
                pltpu.make_async_copy(dst_ref.at[b0 + i], dbs[p],
                                      msem).wait()

                @pl.when(i >= 3)
                def _():
                    pltpu.make_async_copy(rws[p], acc.at[dsc[p]],
                                          ssem).wait()

                pltpu.async_copy(xall_ref.at[gis[p]], rws[p], gsems[p])

            im1 = i - 1
            q = (p + 2) % 3

            @pl.when((im1 >= 0) & (im1 < M))
            def _():
                pltpu.make_async_copy(xall_ref.at[gis[q]], rws[q],
                                      gsems[q]).wait()
                for k in range(EB // L):
                    dsc[q][pl.ds(k * L, L)] = dbs[q][pl.ds(k * L, L)]

                def scale(j, c2):
                    nv = nbs[q][pl.ds(j * L, L)]
                    for k in range(L):
                        e = j * L + k
                        sc = nv[k]
                        for f in range(D // L):
                            rws[q][e, pl.ds(f * L, L)] = (
                                rws[q][e, pl.ds(f * L, L)] * sc)
                    return c2

                lax.fori_loop(0, EB // L, scale, 0)
                pltpu.async_copy(rws[q], acc.at[dsc[q]], ssem, add=True)
                fire_meta(im1 + 3, q)

        return carry

    lax.fori_loop(0, M // 3 + 1, lambda ii, cr: super_it(ii * 3, cr), 0)

    pltpu.make_async_copy(rws[0], acc.at[dsc[0]], ssem).wait()
    pltpu.make_async_copy(rws[1], acc.at[dsc[1]], ssem).wait()
    pltpu.make_async_copy(rws[2], acc.at[dsc[2]], ssem).wait()

    plsc.subcore_barrier()

    @pl.when(s < NBLK)
    def _():
        pltpu.sync_copy(acc.at[pl.ds(s * BN, BN)],
                        out_ref.at[c, pl.ds(s * BN, BN)])


def _agg(xall, gidx2, norm2, dst2, zeros_nd):
    f = pl.kernel(
        _agg_body,
        out_type=jax.ShapeDtypeStruct((NC, N, D), jnp.float32),
        mesh=_mesh,
        compiler_params=_sc_params,
        scratch_types=(
            [pltpu.VMEM_SHARED((N, D), jnp.float32)]
            + [pltpu.VMEM((EB,), jnp.int32) for _ in range(3)]
            + [pltpu.VMEM((EB,), jnp.float32) for _ in range(3)]
            + [pltpu.VMEM((EB,), jnp.int32) for _ in range(6)]
            + [pltpu.VMEM((EB, D), jnp.float32) for _ in range(3)]
            + [pltpu.SemaphoreType.DMA for _ in range(5)]
        ),
    )
    return f(xall, gidx2, norm2, dst2, zeros_nd)


def _l2_body(msg_ref, root_ref, b_ref, w_ref, out_ref, h_scr):
    r = pl.program_id(1)

    @pl.when(r == 0)
    def _():
        h_scr[...] = jnp.maximum(
            msg_ref[0] + msg_ref[1] + root_ref[...] + b_ref[...], 0.0)

    out_ref[...] = jnp.dot(h_scr[...], w_ref[0],
                           preferred_element_type=jnp.float32)


def _k4(msg1, xall1, b1r, W2s):
    return pl.pallas_call(
        _l2_body,
        grid=(NBLK, RW),
        in_specs=[
            pl.BlockSpec((NC, BN, D), lambda nb, r: (0, nb, 0)),
            pl.BlockSpec((BN, D), lambda nb, r: (R * NBLK + nb, 0)),
            pl.BlockSpec((1, D), lambda nb, r: (0, 0)),
            pl.BlockSpec((1, D, D), lambda nb, r: (r, 0, 0)),
        ],
        out_specs=pl.BlockSpec((BN, D), lambda nb, r: (r * NBLK + nb, 0)),
        out_shape=jax.ShapeDtypeStruct((RW * N, D), jnp.float32),
        scratch_shapes=[pltpu.VMEM((BN, D), jnp.float32)],
        compiler_params=pltpu.CompilerParams(
            dimension_semantics=("arbitrary", "arbitrary")),
    )(msg1, xall1, b1r, W2s)


def _fin_body(msg_ref, root_ref, b_ref, batch_ref, emb_ref, sums_ref,
              cnts_ref):
    nb = pl.program_id(0)
    emb = jnp.maximum(
        msg_ref[0] + msg_ref[1] + root_ref[...] + b_ref[...], 0.0)
    emb_ref[...] = emb
    bt = batch_ref[0, 0, :]
    onehot = (bt[None, :] ==
              lax.broadcasted_iota(jnp.int32, (G, BN), 0)).astype(jnp.float32)
    ps = jnp.dot(onehot, emb, preferred_element_type=jnp.float32)
    pc = jnp.sum(onehot, axis=1, keepdims=True) * jnp.ones((1, D), jnp.float32)

    @pl.when(nb == 0)
    def _():
        sums_ref[...] = jnp.zeros_like(sums_ref)
        cnts_ref[...] = jnp.zeros_like(cnts_ref)

    sums_ref[...] += ps
    cnts_ref[...] += pc


def _k6(msg2, xall2, b2r, batch3):
    return pl.pallas_call(
        _fin_body,
        grid=(NBLK,),
        in_specs=[
            pl.BlockSpec((NC, BN, D), lambda nb: (0, nb, 0)),
            pl.BlockSpec((BN, D), lambda nb: (R * NBLK + nb, 0)),
            pl.BlockSpec((1, D), lambda nb: (0, 0)),
            pl.BlockSpec((1, 1, BN), lambda nb: (nb, 0, 0)),
        ],
        out_specs=[
            pl.BlockSpec((BN, D), lambda nb: (nb, 0)),
            pl.BlockSpec((G, D), lambda nb: (0, 0)),
            pl.BlockSpec((G, D), lambda nb: (0, 0)),
        ],
        out_shape=[
            jax.ShapeDtypeStruct((N, D), jnp.float32),
            jax.ShapeDtypeStruct((G, D), jnp.float32),
            jax.ShapeDtypeStruct((G, D), jnp.float32),
        ],
        compiler_params=pltpu.CompilerParams(
            dimension_semantics=("arbitrary",)),
    )(msg2, xall2, b2r, batch3)


def _mlp_body(sums_ref, cnts_ref, wm1_ref, bm1_ref, wm2_ref, bm2_ref,
              logits_ref, probs_ref):
    pooled = sums_ref[...] / jnp.maximum(cnts_ref[...], 1.0)
    v = jnp.dot(pooled, wm1_ref[...],
                preferred_element_type=jnp.float32) + bm1_ref[...]
    hm = jnp.where(v > 0, v, jnp.exp(jnp.minimum(v, 0.0)) - 1.0)
    lg = jnp.dot(hm, wm2_ref[...],
                 preferred_element_type=jnp.float32) + bm2_ref[...]
    logits_ref[...] = lg
    colmask = lax.broadcasted_iota(jnp.int32, (G, D), 1) < OUT
    lgm = jnp.where(colmask, lg, -1e30)
    m = jnp.max(lgm, axis=1, keepdims=True)
    ex = jnp.where(colmask, jnp.exp(lgm - m), 0.0)
    probs_ref[...] = ex / jnp.sum(ex, axis=1, keepdims=True)


def _k7(sums, cnts, Wm1, bm1r, Wm2p, bm2p):
    return pl.pallas_call(
        _mlp_body,
        out_shape=[
            jax.ShapeDtypeStruct((G, D), jnp.float32),
            jax.ShapeDtypeStruct((G, D), jnp.float32),
        ],
    )(sums, cnts, Wm1, bm1r, Wm2p, bm2p)


def kernel(x, W1, root1, b1, W2, root2, b2, Wm1, bm1, Wm2, bm2,
           edge_index, edge_type, batch):
    W1s = jnp.concatenate([W1, root1[None]], axis=0)
    W2s = jnp.concatenate([W2, root2[None]], axis=0)
    zeros_nd = jnp.zeros((N, D), jnp.float32)
    b1r = b1.reshape(1, D)
    b2r = b2.reshape(1, D)
    batch3 = batch.reshape(NBLK, 1, BN)
    Wm2p = jnp.pad(Wm2, ((0, 0), (0, D - OUT)))
    bm2p = jnp.pad(bm2, (0, D - OUT)).reshape(1, D)

    src = edge_index[0]
    dst = edge_index[1]
    pad = EP - E
    spread = (jnp.arange(pad, dtype=jnp.int32) * 97) % N
    src2 = jnp.concatenate([src, spread]).reshape(TBAT, EB)
    dst2 = jnp.concatenate([dst, spread]).reshape(TBAT, EB)
    et2 = jnp.concatenate(
        [edge_type, spread % R]).reshape(TBAT, EB)

    cnt_parts = _k1a(dst, edge_type)
    inv = _k1c(cnt_parts.reshape(NW, CPT // 128, 128)).reshape(CPT)
    gidx2, norm2 = _k1b(src2, dst2, et2, inv)

    xall1 = _k2(x, W1s)
    msg1 = _agg(xall1, gidx2, norm2, dst2, zeros_nd)
    xall2 = _k4(msg1, xall1, b1r, W2s)
    msg2 = _agg(xall2, gidx2, norm2, dst2, zeros_nd)
    emb, sums, cnts = _k6(msg2, xall2, b2r, batch3)
    logits_p, probs_p = _k7(sums, cnts, Wm1, bm1.reshape(1, D), Wm2p, bm2p)
    return (logits_p[:, :OUT], probs_p[:, :OUT], emb)


# speedup vs baseline: 3.2599x; 1.0010x over previous
import jax
import jax.numpy as jnp
from jax import lax
from jax.experimental import pallas as pl
from jax.experimental.pallas import tpu as pltpu
from jax.experimental.pallas import tpu_sc as plsc

N = 10000
E = 320000
R = 8
G = 32
D = 128
OUT = 8
RW = R + 1

NC = 2
NS = 16
NW = NC * NS
L = 16

EB = 128
NBAT = E // EB
NTB_W = 81
TBAT = NW * NTB_W
EP = TBAT * EB
M0 = 81
M1 = 81
CPT = 81920

EPT = E // NW
CH = 2000

BN = 1000
NBLK = N // BN

_mesh = plsc.VectorSubcoreMesh(core_axis_name="c", subcore_axis_name="s")
_sc_params = pltpu.CompilerParams(needs_layout_passes=False,
                                 use_tc_tiling_on_sc=False)


def _worker_ids():
    c = lax.axis_index("c")
    s = lax.axis_index("s")
    return c, s, c * NS + s


def _count_body(dst_ref, et_ref, out_ref, cnt_v, dbuf, ebuf):
    _, _, w = _worker_ids()

    def zero(i, carry):
        cnt_v[pl.ds(i * L, L)] = jnp.zeros((L,), jnp.float32)
        return carry

    lax.fori_loop(0, CPT // L, zero, 0)

    ones = jnp.ones((L,), jnp.float32)

    def chunk(ci, carry):
        base = w * EPT + ci * CH
        pltpu.sync_copy(dst_ref.at[pl.ds(base, CH)], dbuf)
        pltpu.sync_copy(et_ref.at[pl.ds(base, CH)], ebuf)

        def vec(j, c2):
            d16 = dbuf[pl.ds(j * L, L)]
            e16 = ebuf[pl.ds(j * L, L)]
            plsc.addupdate_scatter(cnt_v, [d16 * R + e16], ones)
            return c2

        lax.fori_loop(0, CH // L, vec, 0)
        return carry

    lax.fori_loop(0, EPT // CH, chunk, 0)
    pltpu.sync_copy(cnt_v, out_ref.at[w])


def _k1a(dst, edge_type):
    f = pl.kernel(
        _count_body,
        out_type=jax.ShapeDtypeStruct((NW, CPT), jnp.float32),
        mesh=_mesh,
        compiler_params=_sc_params,
        scratch_types=[
            pltpu.VMEM((CPT,), jnp.float32),
            pltpu.VMEM((CH,), jnp.int32),
            pltpu.VMEM((CH,), jnp.int32),
        ],
    )
    return f(dst, edge_type)


def _inv_body(cnt_ref, out_ref):
    s = jnp.sum(cnt_ref[...], axis=0)
    out_ref[...] = 1.0 / jnp.maximum(s, 1.0)


def _k1c(cnt_parts):
    cb = CPT // 128 // 10
    return pl.pallas_call(
        _inv_body,
        grid=(10,),
        in_specs=[pl.BlockSpec((NW, cb, 128), lambda i: (0, i, 0))],
        out_specs=pl.BlockSpec((cb, 128), lambda i: (i, 0)),
        out_shape=jax.ShapeDtypeStruct((CPT // 128, 128), jnp.float32),
    )(cnt_parts)


def _meta_body(src_ref, dst_ref, et_ref, inv_ref, gidx_ref, norm_ref,
               s2, d2, e2, n2, gsem):
    _, _, w = _worker_ids()
    b0 = w * NTB_W

    pltpu.sync_copy(src_ref.at[pl.ds(b0, NTB_W)], s2)
    pltpu.sync_copy(dst_ref.at[pl.ds(b0, NTB_W)], d2)
    pltpu.sync_copy(et_ref.at[pl.ds(b0, NTB_W)], e2)

    def rowfn(j, carry):
        for k in range(EB // L):
            sl = pl.ds(k * L, L)
            sv = s2[j, sl]
            dv = d2[j, sl]
            ev = e2[j, sl]
            s2[j, sl] = ev * N + sv
            d2[j, sl] = dv * R + ev
        return carry

    lax.fori_loop(0, NTB_W, rowfn, 0)
    pltpu.sync_copy(s2, gidx_ref.at[pl.ds(b0, NTB_W)])

    def fire(i, carry):
        pltpu.async_copy(inv_ref.at[d2.at[i]], n2.at[i], gsem)

        @pl.when(i >= 8)
        def _():
            pltpu.make_async_copy(inv_ref.at[d2.at[i - 8]], n2.at[i - 8],
                                  gsem).wait()

        return carry

    lax.fori_loop(0, NTB_W, fire, 0)
    for j in range(8):
        pltpu.make_async_copy(inv_ref.at[d2.at[NTB_W - 8 + j]],
                              n2.at[NTB_W - 8 + j], gsem).wait()

    jstart = jnp.clip(NBAT - b0, 0, NTB_W)

    def zrow(j, carry):
        for k in range(EB // L):
            n2[j, pl.ds(k * L, L)] = jnp.zeros((L,), jnp.float32)
        return carry

    lax.fori_loop(jstart, NTB_W, zrow, 0)
    pltpu.sync_copy(n2, norm_ref.at[pl.ds(b0, NTB_W)])


def _k1b(src2, dst2, et2, inv):
    f = pl.kernel(
        _meta_body,
        out_type=(
            jax.ShapeDtypeStruct((TBAT, EB), jnp.int32),
            jax.ShapeDtypeStruct((TBAT, EB), jnp.float32),
        ),
        mesh=_mesh,
        compiler_params=_sc_params,
        scratch_types=[
            pltpu.VMEM((NTB_W, EB), jnp.int32),
            pltpu.VMEM((NTB_W, EB), jnp.int32),
            pltpu.VMEM((NTB_W, EB), jnp.int32),
            pltpu.VMEM((NTB_W, EB), jnp.float32),
            pltpu.SemaphoreType.DMA,
        ],
    )
    return f(src2, dst2, et2, inv)


def _xmm_body(x_ref, w_ref, out_ref):
    out_ref[...] = jnp.dot(x_ref[...], w_ref[0],
                           preferred_element_type=jnp.float32)


def _k2(x, Ws):
    return pl.pallas_call(
        _xmm_body,
        grid=(NBLK, RW),
        in_specs=[
            pl.BlockSpec((BN, D), lambda nb, r: (nb, 0)),
            pl.BlockSpec((1, D, D), lambda nb, r: (r, 0, 0)),
        ],
        out_specs=pl.BlockSpec((BN, D), lambda nb, r: (r * NBLK + nb, 0)),
        out_shape=jax.ShapeDtypeStruct((RW * N, D), jnp.float32),
        compiler_params=pltpu.CompilerParams(
            dimension_semantics=("arbitrary", "arbitrary")),
    )(x, Ws)


def _agg_body(xall_ref, gidx_ref, norm_ref, dst_ref, zeros_ref, out_ref,
              acc, gi0, gi1, gi2b, nb0, nb1, nb2b, db0, db1, db2b,
              ds0, ds1, ds2, rows0, rows1, rows2,
              msem, gsem0, gsem1, gsem2, ssem):
    c, s, _ = _worker_ids()
    M = lax.select(c == 0, M0, M1)
    b0 = lax.select(c == 0, s * M0, NS * M0 + s * M1)
    gis = (gi0, gi1, gi2b)
    nbs = (nb0, nb1, nb2b)
    dbs = (db0, db1, db2b)
    dsc = (ds0, ds1, ds2)
    rws = (rows0, rows1, rows2)
    gsems = (gsem0, gsem1, gsem2)

    @pl.when(s == 0)
    def _():
        pltpu.sync_copy(zeros_ref, acc)

    def fire_meta(i, q):
        @pl.when(i < M)
        def _():
            pltpu.async_copy(gidx_ref.at[b0 + i], gis[q], msem)
            pltpu.async_copy(norm_ref.at[b0 + i], nbs[q], msem)
            pltpu.async_copy(dst_ref.at[b0 + i], dbs[q], msem)

    for q in range(3):
        fire_meta(q, q)
    plsc.subcore_barrier()

    def super_it(i0, carry):
        for p in range(3):
            i = i0 + p

            @pl.when(i < M)
            def _():
                pltpu.make_async_copy(gidx_ref.at[b0 + i], gis[p],
                                      msem).wait()
                pltpu.make_async_copy(norm_ref.at[b0 + i], nbs[p],
                                      msem).wait()
                pltpu.make_async_copy(dst_ref.at[b0 + i], dbs[p],
                                      msem).wait()

                @pl.when(i >= 3)
                def _():
                    pltpu.make_async_copy(rws[p], acc.at[dsc[p]],
                                          ssem).wait()

                pltpu.async_copy(xall_ref.at[gis[p]], rws[p], gsems[p])

            im1 = i - 1
            q = (p + 2) % 3

            @pl.when((im1 >= 0) & (im1 < M))
            def _():
                pltpu.make_async_copy(xall_ref.at[gis[q]], rws[q],
                                      gsems[q]).wait()
                for k in range(EB // L):
                    dsc[q][pl.ds(k * L, L)] = dbs[q][pl.ds(k * L, L)]

                def scale(j, c2):
                    nv = nbs[q][pl.ds(j * L, L)]
                    for k in range(L):
                        e = j * L + k
                        sc = nv[k]
                        for f in range(D // L):
                            rws[q][e, pl.ds(f * L, L)] = (
                                rws[q][e, pl.ds(f * L, L)] * sc)
                    return c2

                lax.fori_loop(0, EB // L, scale, 0)
                pltpu.async_copy(rws[q], acc.at[dsc[q]], ssem, add=True)
                fire_meta(im1 + 3, q)

        return carry

    lax.fori_loop(0, M // 3 + 1, lambda ii, cr: super_it(ii * 3, cr), 0)

    pltpu.make_async_copy(rws[0], acc.at[dsc[0]], ssem).wait()
    pltpu.make_async_copy(rws[1], acc.at[dsc[1]], ssem).wait()
    pltpu.make_async_copy(rws[2], acc.at[dsc[2]], ssem).wait()

    plsc.subcore_barrier()

    @pl.when(s < NBLK)
    def _():
        pltpu.sync_copy(acc.at[pl.ds(s * BN, BN)],
                        out_ref.at[c, pl.ds(s * BN, BN)])


def _agg(xall, gidx2, norm2, dst2, zeros_nd):
    f = pl.kernel(
        _agg_body,
        out_type=jax.ShapeDtypeStruct((NC, N, D), jnp.float32),
        mesh=_mesh,
        compiler_params=_sc_params,
        scratch_types=(
            [pltpu.VMEM_SHARED((N, D), jnp.float32)]
            + [pltpu.VMEM((EB,), jnp.int32) for _ in range(3)]
            + [pltpu.VMEM((EB,), jnp.float32) for _ in range(3)]
            + [pltpu.VMEM((EB,), jnp.int32) for _ in range(6)]
            + [pltpu.VMEM((EB, D), jnp.float32) for _ in range(3)]
            + [pltpu.SemaphoreType.DMA for _ in range(5)]
        ),
    )
    return f(xall, gidx2, norm2, dst2, zeros_nd)


def _l2_body(msg_ref, root_ref, b_ref, w_ref, out_ref, h_scr):
    r = pl.program_id(1)

    @pl.when(r == 0)
    def _():
        h_scr[...] = jnp.maximum(
            msg_ref[0] + msg_ref[1] + root_ref[...] + b_ref[...], 0.0)

    out_ref[...] = jnp.dot(h_scr[...], w_ref[0],
                           preferred_element_type=jnp.float32)


def _k4(msg1, xall1, b1r, W2s):
    return pl.pallas_call(
        _l2_body,
        grid=(NBLK, RW),
        in_specs=[
            pl.BlockSpec((NC, BN, D), lambda nb, r: (0, nb, 0)),
            pl.BlockSpec((BN, D), lambda nb, r: (R * NBLK + nb, 0)),
            pl.BlockSpec((1, D), lambda nb, r: (0, 0)),
            pl.BlockSpec((1, D, D), lambda nb, r: (r, 0, 0)),
        ],
        out_specs=pl.BlockSpec((BN, D), lambda nb, r: (r * NBLK + nb, 0)),
        out_shape=jax.ShapeDtypeStruct((RW * N, D), jnp.float32),
        scratch_shapes=[pltpu.VMEM((BN, D), jnp.float32)],
        compiler_params=pltpu.CompilerParams(
            dimension_semantics=("arbitrary", "arbitrary")),
    )(msg1, xall1, b1r, W2s)


def _fin_body(msg_ref, root_ref, b_ref, batch_ref, emb_ref, sums_ref,
              cnts_ref):
    nb = pl.program_id(0)
    emb = jnp.maximum(
        msg_ref[0] + msg_ref[1] + root_ref[...] + b_ref[...], 0.0)
    emb_ref[...] = emb
    bt = batch_ref[0, 0, :]
    onehot = (bt[None, :] ==
              lax.broadcasted_iota(jnp.int32, (G, BN), 0)).astype(jnp.float32)
    ps = jnp.dot(onehot, emb, preferred_element_type=jnp.float32)
    pc = jnp.sum(onehot, axis=1, keepdims=True) * jnp.ones((1, D), jnp.float32)

    @pl.when(nb == 0)
    def _():
        sums_ref[...] = jnp.zeros_like(sums_ref)
        cnts_ref[...] = jnp.zeros_like(cnts_ref)

    sums_ref[...] += ps
    cnts_ref[...] += pc


def _k6(msg2, xall2, b2r, batch3):
    return pl.pallas_call(
        _fin_body,
        grid=(NBLK,),
        in_specs=[
            pl.BlockSpec((NC, BN, D), lambda nb: (0, nb, 0)),
            pl.BlockSpec((BN, D), lambda nb: (R * NBLK + nb, 0)),
            pl.BlockSpec((1, D), lambda nb: (0, 0)),
            pl.BlockSpec((1, 1, BN), lambda nb: (nb, 0, 0)),
        ],
        out_specs=[
            pl.BlockSpec((BN, D), lambda nb: (nb, 0)),
            pl.BlockSpec((G, D), lambda nb: (0, 0)),
            pl.BlockSpec((G, D), lambda nb: (0, 0)),
        ],
        out_shape=[
            jax.ShapeDtypeStruct((N, D), jnp.float32),
            jax.ShapeDtypeStruct((G, D), jnp.float32),
            jax.ShapeDtypeStruct((G, D), jnp.float32),
        ],
        compiler_params=pltpu.CompilerParams(
            dimension_semantics=("arbitrary",)),
    )(msg2, xall2, b2r, batch3)


def _mlp_body(sums_ref, cnts_ref, wm1_ref, bm1_ref, wm2_ref, bm2_ref,
              logits_ref, probs_ref):
    pooled = sums_ref[...] / jnp.maximum(cnts_ref[...], 1.0)
    v = jnp.dot(pooled, wm1_ref[...],
                preferred_element_type=jnp.float32) + bm1_ref[...]
    hm = jnp.where(v > 0, v, jnp.exp(jnp.minimum(v, 0.0)) - 1.0)
    lg = jnp.dot(hm, wm2_ref[...],
                 preferred_element_type=jnp.float32) + bm2_ref[...]
    logits_ref[...] = lg
    colmask = lax.broadcasted_iota(jnp.int32, (G, D), 1) < OUT
    lgm = jnp.where(colmask, lg, -1e30)
    m = jnp.max(lgm, axis=1, keepdims=True)
    ex = jnp.where(colmask, jnp.exp(lgm - m), 0.0)
    probs_ref[...] = ex / jnp.sum(ex, axis=1, keepdims=True)


def _k7(sums, cnts, Wm1, bm1r, Wm2p, bm2p):
    return pl.pallas_call(
        _mlp_body,
        out_shape=[
            jax.ShapeDtypeStruct((G, D), jnp.float32),
            jax.ShapeDtypeStruct((G, D), jnp.float32),
        ],
    )(sums, cnts, Wm1, bm1r, Wm2p, bm2p)


def kernel(x, W1, root1, b1, W2, root2, b2, Wm1, bm1, Wm2, bm2,
           edge_index, edge_type, batch):
    W1s = jnp.concatenate([W1, root1[None]], axis=0)
    W2s = jnp.concatenate([W2, root2[None]], axis=0)
    zeros_nd = jnp.zeros((N, D), jnp.float32)
    b1r = b1.reshape(1, D)
    b2r = b2.reshape(1, D)
    batch3 = batch.reshape(NBLK, 1, BN)
    Wm2p = jnp.pad(Wm2, ((0, 0), (0, D - OUT)))
    bm2p = jnp.pad(bm2, (0, D - OUT)).reshape(1, D)

    src = edge_index[0]
    dst = edge_index[1]
    pad = EP - E
    spread = (jnp.arange(pad, dtype=jnp.int32) * 97) % N
    src2 = jnp.concatenate([src, spread]).reshape(TBAT, EB)
    dst2 = jnp.concatenate([dst, spread]).reshape(TBAT, EB)
    et2 = jnp.concatenate(
        [edge_type, spread % R]).reshape(TBAT, EB)

    xall1 = _k2(x, W1s)
    cnt_parts = _k1a(dst, edge_type)
    inv = _k1c(cnt_parts.reshape(NW, CPT // 128, 128)).reshape(CPT)
    gidx2, norm2 = _k1b(src2, dst2, et2, inv)

    msg1 = _agg(xall1, gidx2, norm2, dst2, zeros_nd)
    xall2 = _k4(msg1, xall1, b1r, W2s)
    msg2 = _agg(xall2, gidx2, norm2, dst2, zeros_nd)
    emb, sums, cnts = _k6(msg2, xall2, b2r, batch3)
    logits_p, probs_p = _k7(sums, cnts, Wm1, bm1.reshape(1, D), Wm2p, bm2p)
    return (logits_p[:, :OUT], probs_p[:, :OUT], emb)
